# baseline (speedup 1.0000x reference)
import functools

import jax
import jax.numpy as jnp
from jax import lax
from jax.experimental import pallas as pl
from jax.experimental.pallas import tpu as pltpu
from jax.experimental.pallas import tpu_sc as plsc

N = 16384
C = 1000
NC = 2
NS = 16
NW = NC * NS
ROWS_PER_SUB = N // NW
BLK = 128
BLOCKS = ROWS_PER_SUB // BLK
LANE = 16


def _body(probT_hbm, tgt_hbm, rew_hbm, part_hbm,
          tgt_v, rew_v, val_v, acc_v,
          sem_s, sem_a, sem_b, sem_c, sem_d):
    wid = lax.axis_index("c") * NS + lax.axis_index("s")
    base = wid * ROWS_PER_SUB

    cp_t = pltpu.make_async_copy(
        tgt_hbm.at[pl.ds(base, ROWS_PER_SUB)], tgt_v, sem_s)
    cp_r = pltpu.make_async_copy(
        rew_hbm.at[pl.ds(base, ROWS_PER_SUB)], rew_v, sem_s)
    cp_t.start()
    cp_r.start()
    cp_t.wait()

    sems = (sem_a, sem_b, sem_c, sem_d)
    NBUF = len(sems)

    def fire(b):
        idx_ref = tgt_v.at[pl.ds(b * BLK, BLK)]
        cp = pltpu.make_async_copy(
            probT_hbm.at[idx_ref, pl.ds(base + b * BLK, BLK)],
            val_v.at[b % NBUF], sems[b % NBUF])
        cp.start()
        return cp

    iota = lax.iota(jnp.int32, LANE)
    onehot = [jnp.where(iota == i, 1.0, 0.0) for i in range(LANE)]
    cps = [fire(b) for b in range(min(NBUF, BLOCKS))]
    cp_r.wait()
    acc = jnp.zeros((LANE,), jnp.float32)
    for b in range(BLOCKS):
        cps[b].wait()
        for k in range(BLK // LANE):
            d0 = jnp.zeros((LANE,), jnp.float32)
            d1 = jnp.zeros((LANE,), jnp.float32)
            for i in range(0, LANE, 2):
                d0 = d0 + val_v[b % NBUF, k * LANE + i,
                                pl.ds(k * LANE, LANE)] * onehot[i]
                d1 = d1 + val_v[b % NBUF, k * LANE + i + 1,
                                pl.ds(k * LANE, LANE)] * onehot[i + 1]
            r = rew_v[pl.ds(b * BLK + k * LANE, LANE)]
            acc = acc + jnp.exp(d0 + d1) * r
        if b + NBUF < BLOCKS:
            cps.append(fire(b + NBUF))
    acc_v[...] = acc
    pltpu.sync_copy(acc_v, part_hbm.at[wid])


def _tc_reduce(part_ref, out_ref):
    out_ref[0, 0] = jnp.sum(part_ref[...]) * (1.0 / N)


@jax.jit
def _ganloss_sc(probT, target, reward):
    mesh = plsc.VectorSubcoreMesh(core_axis_name="c", subcore_axis_name="s",
                                  num_cores=NC)
    k = functools.partial(
        pl.kernel,
        mesh=mesh,
        out_type=jax.ShapeDtypeStruct((NW, LANE), jnp.float32),
        scratch_types=[
            pltpu.VMEM((ROWS_PER_SUB,), jnp.int32),
            pltpu.VMEM((ROWS_PER_SUB,), jnp.float32),
            pltpu.VMEM((4, BLK, BLK), jnp.float32),
            pltpu.VMEM((LANE,), jnp.float32),
            pltpu.SemaphoreType.DMA,
            pltpu.SemaphoreType.DMA,
            pltpu.SemaphoreType.DMA,
            pltpu.SemaphoreType.DMA,
            pltpu.SemaphoreType.DMA,
        ],
    )(_body)
    part = k(probT, target, reward)
    out = pl.pallas_call(
        _tc_reduce,
        out_shape=jax.ShapeDtypeStruct((1, 1), jnp.float32),
        out_specs=pl.BlockSpec(memory_space=pltpu.SMEM),
    )(part)
    return out[0, 0]


def kernel(prob, target, reward):
    return _ganloss_sc(prob.T, target.astype(jnp.int32),
                       reward.astype(jnp.float32))


# speedup vs baseline: 1.0147x; 1.0147x over previous
import functools

import jax
import jax.numpy as jnp
from jax import lax
from jax.experimental import pallas as pl
from jax.experimental.pallas import tpu as pltpu
from jax.experimental.pallas import tpu_sc as plsc

N = 16384
C = 1000
NC = 2
NS = 16
NW = NC * NS
ROWS_PER_SUB = N // NW
BLK = 128
BLOCKS = ROWS_PER_SUB // BLK
LANE = 16


def _body(probT_hbm, tgt_hbm, rew_hbm, part_hbm,
          tgt_v, rew_v, val_v, acc_v,
          sem_s, sem_a, sem_b):
    wid = lax.axis_index("c") * NS + lax.axis_index("s")
    base = wid * ROWS_PER_SUB

    cp_t = pltpu.make_async_copy(
        tgt_hbm.at[pl.ds(base, ROWS_PER_SUB)], tgt_v, sem_s)
    cp_r = pltpu.make_async_copy(
        rew_hbm.at[pl.ds(base, ROWS_PER_SUB)], rew_v, sem_s)
    cp_t.start()
    cp_r.start()
    cp_t.wait()

    sems = (sem_a, sem_b)
    NBUF = len(sems)

    def fire(b):
        idx_ref = tgt_v.at[pl.ds(b * BLK, BLK)]
        cp = pltpu.make_async_copy(
            probT_hbm.at[idx_ref, pl.ds(base + b * BLK, BLK)],
            val_v.at[b % NBUF], sems[b % NBUF])
        cp.start()
        return cp

    iota = lax.iota(jnp.int32, LANE)
    onehot = [jnp.where(iota == i, 1.0, 0.0) for i in range(LANE)]
    cps = [fire(b) for b in range(min(NBUF, BLOCKS))]
    cp_r.wait()
    acc = jnp.zeros((LANE,), jnp.float32)
    for b in range(BLOCKS):
        cps[b].wait()
        for k in range(BLK // LANE):
            d0 = jnp.zeros((LANE,), jnp.float32)
            d1 = jnp.zeros((LANE,), jnp.float32)
            for i in range(0, LANE, 2):
                d0 = d0 + val_v[b % NBUF, k * LANE + i,
                                pl.ds(k * LANE, LANE)] * onehot[i]
                d1 = d1 + val_v[b % NBUF, k * LANE + i + 1,
                                pl.ds(k * LANE, LANE)] * onehot[i + 1]
            r = rew_v[pl.ds(b * BLK + k * LANE, LANE)]
            acc = acc + jnp.exp(d0 + d1) * r
        if b + NBUF < BLOCKS:
            cps.append(fire(b + NBUF))
    acc_v[...] = acc
    pltpu.sync_copy(acc_v, part_hbm.at[wid])


def _tc_reduce(part_ref, out_ref):
    out_ref[0, 0] = jnp.sum(part_ref[...]) * (1.0 / N)


@jax.jit
def _ganloss_sc(probT, target, reward):
    mesh = plsc.VectorSubcoreMesh(core_axis_name="c", subcore_axis_name="s",
                                  num_cores=NC)
    k = functools.partial(
        pl.kernel,
        mesh=mesh,
        out_type=jax.ShapeDtypeStruct((NW, LANE), jnp.float32),
        scratch_types=[
            pltpu.VMEM((ROWS_PER_SUB,), jnp.int32),
            pltpu.VMEM((ROWS_PER_SUB,), jnp.float32),
            pltpu.VMEM((2, BLK, BLK), jnp.float32),
            pltpu.VMEM((LANE,), jnp.float32),
            pltpu.SemaphoreType.DMA,
            pltpu.SemaphoreType.DMA,
            pltpu.SemaphoreType.DMA,
        ],
    )(_body)
    part = k(probT, target, reward)
    out = pl.pallas_call(
        _tc_reduce,
        out_shape=jax.ShapeDtypeStruct((1, 1), jnp.float32),
        out_specs=pl.BlockSpec(memory_space=pltpu.SMEM),
    )(part)
    return out[0, 0]


def kernel(prob, target, reward):
    return _ganloss_sc(prob.T, target.astype(jnp.int32),
                       reward.astype(jnp.float32))


# speedup vs baseline: 1.0579x; 1.0426x over previous
import functools

import jax
import jax.numpy as jnp
from jax import lax
from jax.experimental import pallas as pl
from jax.experimental.pallas import tpu as pltpu
from jax.experimental.pallas import tpu_sc as plsc

N = 16384
C = 1000
NC = 2
NS = 16
NW = NC * NS
ROWS_PER_SUB = N // NW
BLK = 128
BLOCKS = ROWS_PER_SUB // BLK
LANE = 16


def _body(probT_hbm, tgt_hbm, rew_hbm, part_hbm,
          tgt_v, rew_v, val_v, acc_v,
          sem_s, sem_a, sem_b):
    wid = lax.axis_index("c") * NS + lax.axis_index("s")
    base = wid * ROWS_PER_SUB

    cp_t = pltpu.make_async_copy(
        tgt_hbm.at[pl.ds(base, ROWS_PER_SUB)], tgt_v, sem_s)
    cp_r = pltpu.make_async_copy(
        rew_hbm.at[pl.ds(base, ROWS_PER_SUB)], rew_v, sem_s)
    cp_t.start()
    cp_r.start()
    cp_t.wait()

    sems = (sem_a, sem_b)
    NBUF = len(sems)

    def fire(b):
        idx_ref = tgt_v.at[pl.ds(b * BLK, BLK)]
        cp = pltpu.make_async_copy(
            probT_hbm.at[idx_ref, pl.ds(base + b * BLK, BLK)],
            val_v.at[b % NBUF], sems[b % NBUF])
        cp.start()
        return cp

    iota = lax.iota(jnp.int32, LANE)
    cps = [fire(b) for b in range(min(NBUF, BLOCKS))]
    cp_r.wait()
    acc = jnp.zeros((LANE,), jnp.float32)
    for b in range(BLOCKS):
        cps[b].wait()

        def chunk(k, acc):
            def row2(i, d):
                d0, d1 = d
                r0 = val_v[b % NBUF, k * LANE + 2 * i, pl.ds(k * LANE, LANE)]
                r1 = val_v[b % NBUF, k * LANE + 2 * i + 1,
                           pl.ds(k * LANE, LANE)]
                d0 = jnp.where(iota == 2 * i, r0, d0)
                d1 = jnp.where(iota == 2 * i + 1, r1, d1)
                return (d0, d1)

            zero = jnp.zeros((LANE,), jnp.float32)
            d0, d1 = lax.fori_loop(0, LANE // 2, row2, (zero, zero))
            r = rew_v[pl.ds(b * BLK + k * LANE, LANE)]
            return acc + jnp.exp(d0 + d1) * r

        acc = lax.fori_loop(0, BLK // LANE, chunk, acc)
        if b + NBUF < BLOCKS:
            cps.append(fire(b + NBUF))
    acc_v[...] = acc
    pltpu.sync_copy(acc_v, part_hbm.at[wid])


def _tc_reduce(part_ref, out_ref):
    out_ref[0, 0] = jnp.sum(part_ref[...]) * (1.0 / N)


@jax.jit
def _ganloss_sc(probT, target, reward):
    mesh = plsc.VectorSubcoreMesh(core_axis_name="c", subcore_axis_name="s",
                                  num_cores=NC)
    k = functools.partial(
        pl.kernel,
        mesh=mesh,
        out_type=jax.ShapeDtypeStruct((NW, LANE), jnp.float32),
        scratch_types=[
            pltpu.VMEM((ROWS_PER_SUB,), jnp.int32),
            pltpu.VMEM((ROWS_PER_SUB,), jnp.float32),
            pltpu.VMEM((2, BLK, BLK), jnp.float32),
            pltpu.VMEM((LANE,), jnp.float32),
            pltpu.SemaphoreType.DMA,
            pltpu.SemaphoreType.DMA,
            pltpu.SemaphoreType.DMA,
        ],
    )(_body)
    part = k(probT, target, reward)
    out = pl.pallas_call(
        _tc_reduce,
        out_shape=jax.ShapeDtypeStruct((1, 1), jnp.float32),
        out_specs=pl.BlockSpec(memory_space=pltpu.SMEM),
    )(part)
    return out[0, 0]


def kernel(prob, target, reward):
    return _ganloss_sc(prob.T, target.astype(jnp.int32),
                       reward.astype(jnp.float32))
